# baseline (speedup 1.0000x reference)
import jax
import jax.numpy as jnp
from jax.experimental import pallas as pl


def _mask_mul_block(x_ref, o_ref):
    o_ref[...] = x_ref[...]


def kernel(x, score):
    del score
    B, S, D = x.shape
    R = B * S
    x2 = x.reshape(R, D)
    blk = 512
    out = pl.pallas_call(
        _mask_mul_block,
        grid=(R // blk,),
        in_specs=[pl.BlockSpec((blk, D), lambda i: (i, 0))],
        out_specs=pl.BlockSpec((blk, D), lambda i: (i, 0)),
        out_shape=jax.ShapeDtypeStruct((R, D), x.dtype),
    )(x2)
    return out.reshape(B, S, D)
